# baseline (speedup 1.0000x reference)
import jax
import jax.numpy as jnp
from jax import lax
from jax.experimental import pallas as pl
from jax.experimental.pallas import tpu as pltpu
from jax.experimental.pallas import tpu_sc as plsc

M = 262144
D = 128
B = 16384
LOG2B = 14

NC = 2
NS = 16
L = 16

TC_BLOCK = 1024


def _tc_norm_body(mem_ref, out_ref):
    x = mem_ref[...]
    normsq = jnp.sum(x * x, axis=1, keepdims=True)
    out_ref[...] = x * (1.0 / jnp.maximum(jnp.sqrt(normsq), 1e-12))


def _tc_normalize(mem, interpret=False):
    return pl.pallas_call(
        _tc_norm_body,
        grid=(M // TC_BLOCK,),
        in_specs=[pl.BlockSpec((TC_BLOCK, D), lambda i: (i, 0))],
        out_specs=pl.BlockSpec((TC_BLOCK, D), lambda i: (i, 0)),
        out_shape=jax.ShapeDtypeStruct((M, D), jnp.float32),
        interpret=interpret,
    )(mem)


EPT = B // NS
G = 128
NG_ACC = EPT // G
HALF = B // 2
DUMP = HALF
S_ROWS = HALF + 8
COMB_ROWS = B + 8


def _mesh():
    return plsc.VectorSubcoreMesh(
        core_axis_name="c", subcore_axis_name="s", num_cores=NC,
        num_subcores=NS,
    )


def _rsqrt16(x):
    i = plsc.bitcast(x, jnp.int32)
    i = jnp.int32(0x5F3759DF) - (i >> 1)
    y = plsc.bitcast(i, jnp.float32)
    for _ in range(3):
        y = y * (1.5 - 0.5 * x * y * y)
    return y


def _acc_body(sidx_hbm, order2d_hbm, val_hbm, zeros_hbm, comb_hbm,
              S, sidx_v, order_sc, slots_sc, tgt_sc, vbuf, cbuf):
    c = lax.axis_index("c")
    s = lax.axis_index("s")

    pltpu.sync_copy(sidx_hbm, sidx_v)
    pltpu.sync_copy(order2d_hbm.at[pl.ds(s * (EPT // G), EPT // G)],
                    order_sc)

    for g in range(HALF // G // NS):
        pltpu.sync_copy(zeros_hbm,
                        S.at[pl.ds(s * (HALF // NS) + g * G, G)])

    @pl.when(s == 0)
    def _():
        pltpu.sync_copy(zeros_hbm.at[pl.ds(0, S_ROWS - HALF)],
                        S.at[pl.ds(HALF, S_ROWS - HALF)])

    def slot_body(v, _):
        pos_g = s * EPT + v * L + lax.iota(jnp.int32, L)
        x = sidx_v[pl.ds(s * EPT + v * L, L)]
        rep = jnp.zeros((L,), jnp.int32)
        for k in (1 << p for p in reversed(range(LOG2B))):
            cand = rep + jnp.int32(k)
            below = plsc.load_gather(sidx_v, [cand - 1]) < x
            rep = jnp.where(below, cand, rep)
        in_half = (rep >> 13) == c
        g = v >> 3
        off = (v & 7) * L
        slots_sc[g, pl.ds(off, L)] = jnp.where(
            in_half, rep - c * HALF, jnp.int32(DUMP))
        tgt_sc[g, pl.ds(off, L)] = jnp.where(in_half, pos_g, jnp.int32(B))
        return 0

    lax.fori_loop(0, EPT // L, slot_body, 0)

    plsc.subcore_barrier()

    for g in range(NG_ACC):
        pltpu.sync_copy(val_hbm.at[order_sc.at[g]], vbuf)
        pltpu.sync_copy(vbuf, S.at[slots_sc.at[g]], add=True)

    plsc.subcore_barrier()

    for g in range(NG_ACC):
        pltpu.sync_copy(S.at[slots_sc.at[g]], cbuf)
        pltpu.sync_copy(cbuf, comb_hbm.at[tgt_sc.at[g]])


def _make_acc(interpret=False):
    return pl.kernel(
        _acc_body,
        out_type=jax.ShapeDtypeStruct((COMB_ROWS, D), jnp.float32),
        mesh=_mesh(),
        compiler_params=pltpu.CompilerParams(needs_layout_passes=False),
        interpret=interpret,
        scratch_types=[
            pltpu.VMEM_SHARED((S_ROWS, D), jnp.float32),
            pltpu.VMEM((B,), jnp.int32),
            pltpu.VMEM((NG_ACC, G), jnp.int32),
            pltpu.VMEM((NG_ACC, G), jnp.int32),
            pltpu.VMEM((NG_ACC, G), jnp.int32),
            pltpu.VMEM((G, D), jnp.float32),
            pltpu.VMEM((G, D), jnp.float32),
        ],
    )


EPW = B // (NC * NS)
NG_FIX = EPW // G


def _fix_body(mem_hbm, sidx2d_hbm, comb_hbm, out_ref, sidx_sc, mbuf, cbuf):
    c = lax.axis_index("c")
    s = lax.axis_index("s")
    wid = s * NC + c
    base = wid * EPW

    pltpu.sync_copy(sidx2d_hbm.at[pl.ds(wid * NG_FIX, NG_FIX)], sidx_sc)

    for g in range(NG_FIX):
        pltpu.sync_copy(mem_hbm.at[sidx_sc.at[g]], mbuf)
        pltpu.sync_copy(comb_hbm.at[pl.ds(base + g * G, G)], cbuf)

        for blk in range(G // L):
            rows = lax.iota(jnp.int32, L) + blk * L

            def col_accum(k, ns):
                kv = jnp.full((L,), k, jnp.int32)
                u = (plsc.load_gather(mbuf, [rows, kv])
                     + plsc.load_gather(cbuf, [rows, kv]))
                plsc.store_scatter(mbuf, [rows, kv], u)
                return ns + u * u

            normsq = lax.fori_loop(0, D, col_accum,
                                   jnp.zeros((L,), jnp.float32))
            inv = _rsqrt16(jnp.maximum(normsq, 1e-24))

            def col_scale(k, _):
                kv = jnp.full((L,), k, jnp.int32)
                u = plsc.load_gather(mbuf, [rows, kv])
                plsc.store_scatter(mbuf, [rows, kv], u * inv)
                return 0

            lax.fori_loop(0, D, col_scale, 0)

        pltpu.sync_copy(mbuf, out_ref.at[sidx_sc.at[g]])


def _make_fix(interpret=False):
    return pl.kernel(
        _fix_body,
        mesh=_mesh(),
        compiler_params=pltpu.CompilerParams(needs_layout_passes=False),
        interpret=interpret,
        scratch_types=[
            pltpu.VMEM((NG_FIX, G), jnp.int32),
            pltpu.VMEM((G, D), jnp.float32),
            pltpu.VMEM((G, D), jnp.float32),
        ],
    )


def _kernel_impl(mem, idx, val, interpret=False):
    idx32 = idx.astype(jnp.int32)
    sidx, order = lax.sort_key_val(idx32, jnp.arange(B, dtype=jnp.int32))
    order2d = order.reshape(B // G, G)
    sidx2d = sidx.reshape(B // G, G)
    zeros = jnp.zeros((G, D), jnp.float32)

    out1 = _tc_normalize(mem, interpret)
    comb = _make_acc(interpret)(sidx, order2d, val, zeros)

    oref = jax.new_ref(out1)
    _make_fix(interpret)(mem, sidx2d, comb, oref)
    return oref[...]


@jax.jit
def kernel(mem, idx, val):
    return _kernel_impl(mem, idx, val)


# speedup vs baseline: 3.5220x; 3.5220x over previous
import jax
import jax.numpy as jnp
from jax import lax
from jax.experimental import pallas as pl
from jax.experimental.pallas import tpu as pltpu
from jax.experimental.pallas import tpu_sc as plsc

M = 262144
D = 128
B = 16384
LOG2B = 14

NC = 2
NS = 16
L = 16

TC_BLOCK = 1024


def _tc_norm_body(mem_ref, out_ref):
    x = mem_ref[...]
    normsq = jnp.sum(x * x, axis=1, keepdims=True)
    out_ref[...] = x * (1.0 / jnp.maximum(jnp.sqrt(normsq), 1e-12))


def _tc_normalize(mem, interpret=False):
    return pl.pallas_call(
        _tc_norm_body,
        grid=(M // TC_BLOCK,),
        in_specs=[pl.BlockSpec((TC_BLOCK, D), lambda i: (i, 0))],
        out_specs=pl.BlockSpec((TC_BLOCK, D), lambda i: (i, 0)),
        out_shape=jax.ShapeDtypeStruct((M, D), jnp.float32),
        interpret=interpret,
    )(mem)


EPT = B // NS
G = 128
NG_ACC = EPT // G
HALF = B // 2
DUMP = HALF
S_ROWS = HALF + 8
COMB_ROWS = B + 8


def _mesh():
    return plsc.VectorSubcoreMesh(
        core_axis_name="c", subcore_axis_name="s", num_cores=NC,
        num_subcores=NS,
    )


def _rsqrt16(x):
    i = plsc.bitcast(x, jnp.int32)
    i = jnp.int32(0x5F3759DF) - (i >> 1)
    y = plsc.bitcast(i, jnp.float32)
    for _ in range(3):
        y = y * (1.5 - 0.5 * x * y * y)
    return y


PAD = EPT + L


def _acc_body(sidx_hbm, order_hbm, val_hbm, zeros_hbm, comb_hbm,
              S, sidx_v, order_v, slots_f, order_f, tgt_f,
              slots2, order2, tgt2, vbuf, cbuf):
    c = lax.axis_index("c")
    s = lax.axis_index("s")

    pltpu.sync_copy(sidx_hbm, sidx_v)
    pltpu.sync_copy(order_hbm.at[pl.ds(s * EPT, EPT)], order_v)

    for g in range(HALF // G // NS):
        pltpu.sync_copy(zeros_hbm,
                        S.at[pl.ds(s * (HALF // NS) + g * G, G)])

    @pl.when(s == 0)
    def _():
        pltpu.sync_copy(zeros_hbm.at[pl.ds(0, S_ROWS - HALF)],
                        S.at[pl.ds(HALF, S_ROWS - HALF)])

    def pad_body(v, _):
        slots_f[pl.ds(v * L, L)] = jnp.full((L,), DUMP, jnp.int32)
        order_f[pl.ds(v * L, L)] = jnp.zeros((L,), jnp.int32)
        tgt_f[pl.ds(v * L, L)] = jnp.full((L,), B, jnp.int32)
        return 0

    lax.fori_loop(0, PAD // L, pad_body, 0)

    def slot_body(v, cnt):
        pos_g = s * EPT + v * L + lax.iota(jnp.int32, L)
        x = sidx_v[pl.ds(s * EPT + v * L, L)]
        ov = order_v[pl.ds(v * L, L)]
        rep = jnp.zeros((L,), jnp.int32)
        for k in (1 << p for p in reversed(range(LOG2B))):
            cand = rep + jnp.int32(k)
            below = plsc.load_gather(sidx_v, [cand - 1]) < x
            rep = jnp.where(below, cand, rep)
        in_half = (rep >> 13) == c
        plsc.store_compressed(slots_f.at[pl.ds(cnt, L)],
                              rep - c * HALF, mask=in_half)
        plsc.store_compressed(order_f.at[pl.ds(cnt, L)], ov, mask=in_half)
        plsc.store_compressed(tgt_f.at[pl.ds(cnt, L)], pos_g, mask=in_half)
        return cnt + jnp.sum(in_half.astype(jnp.int32))

    k_cnt = lax.fori_loop(0, EPT // L, slot_body, jnp.int32(0))
    ng = (k_cnt + (G - 1)) >> 7

    def copy_body(v, _):
        r = v >> 3
        off = (v & 7) * L
        slots2[r, pl.ds(off, L)] = slots_f[pl.ds(v * L, L)]
        order2[r, pl.ds(off, L)] = order_f[pl.ds(v * L, L)]
        tgt2[r, pl.ds(off, L)] = tgt_f[pl.ds(v * L, L)]
        return 0

    lax.fori_loop(0, EPT // L, copy_body, 0)

    plsc.subcore_barrier()

    def acc_grp(g, _):
        pltpu.sync_copy(val_hbm.at[order2.at[g]], vbuf)
        pltpu.sync_copy(vbuf, S.at[slots2.at[g]], add=True)
        return 0

    lax.fori_loop(0, ng, acc_grp, 0)

    plsc.subcore_barrier()

    def out_grp(g, _):
        pltpu.sync_copy(S.at[slots2.at[g]], cbuf)
        pltpu.sync_copy(cbuf, comb_hbm.at[tgt2.at[g]])
        return 0

    lax.fori_loop(0, ng, out_grp, 0)


def _make_acc(interpret=False):
    return pl.kernel(
        _acc_body,
        out_type=jax.ShapeDtypeStruct((COMB_ROWS, D), jnp.float32),
        mesh=_mesh(),
        compiler_params=pltpu.CompilerParams(needs_layout_passes=False),
        interpret=interpret,
        scratch_types=[
            pltpu.VMEM_SHARED((S_ROWS, D), jnp.float32),
            pltpu.VMEM((B,), jnp.int32),
            pltpu.VMEM((EPT,), jnp.int32),
            pltpu.VMEM((PAD,), jnp.int32),
            pltpu.VMEM((PAD,), jnp.int32),
            pltpu.VMEM((PAD,), jnp.int32),
            pltpu.VMEM((NG_ACC, G), jnp.int32),
            pltpu.VMEM((NG_ACC, G), jnp.int32),
            pltpu.VMEM((NG_ACC, G), jnp.int32),
            pltpu.VMEM((G, D), jnp.float32),
            pltpu.VMEM((G, D), jnp.float32),
        ],
    )


EPW = B // (NC * NS)
NG_FIX = EPW // G


def _fix_body(mem_hbm, sidx2d_hbm, comb_hbm, out_ref, sidx_sc, mbuf, cbuf):
    c = lax.axis_index("c")
    s = lax.axis_index("s")
    wid = s * NC + c
    base = wid * EPW

    pltpu.sync_copy(sidx2d_hbm.at[pl.ds(wid * NG_FIX, NG_FIX)], sidx_sc)

    for g in range(NG_FIX):
        pltpu.sync_copy(mem_hbm.at[sidx_sc.at[g]], mbuf)
        pltpu.sync_copy(comb_hbm.at[pl.ds(base + g * G, G)], cbuf)

        def row_body(r, _):
            us = []
            ns = jnp.zeros((L,), jnp.float32)
            for k in range(D // L):
                u = mbuf[r, pl.ds(k * L, L)] + cbuf[r, pl.ds(k * L, L)]
                us.append(u)
                ns = ns + u * u
            tot = jnp.full((L,), jnp.sum(ns), jnp.float32)
            inv = _rsqrt16(jnp.maximum(tot, 1e-24))
            for k in range(D // L):
                mbuf[r, pl.ds(k * L, L)] = us[k] * inv
            return 0

        lax.fori_loop(0, G, row_body, 0, unroll=2)

        pltpu.sync_copy(mbuf, out_ref.at[sidx_sc.at[g]])


def _make_fix(interpret=False):
    return pl.kernel(
        _fix_body,
        mesh=_mesh(),
        compiler_params=pltpu.CompilerParams(needs_layout_passes=False),
        interpret=interpret,
        scratch_types=[
            pltpu.VMEM((NG_FIX, G), jnp.int32),
            pltpu.VMEM((G, D), jnp.float32),
            pltpu.VMEM((G, D), jnp.float32),
        ],
    )


def _kernel_impl(mem, idx, val, interpret=False):
    idx32 = idx.astype(jnp.int32)
    sidx, order = lax.sort_key_val(idx32, jnp.arange(B, dtype=jnp.int32))
    sidx2d = sidx.reshape(B // G, G)
    zeros = jnp.zeros((G, D), jnp.float32)

    out1 = _tc_normalize(mem, interpret)
    comb = _make_acc(interpret)(sidx, order, val, zeros)

    oref = jax.new_ref(out1)
    _make_fix(interpret)(mem, sidx2d, comb, oref)
    return oref[...]


@jax.jit
def kernel(mem, idx, val):
    return _kernel_impl(mem, idx, val)


# speedup vs baseline: 3.6239x; 1.0289x over previous
import jax
import jax.numpy as jnp
from jax import lax
from jax.experimental import pallas as pl
from jax.experimental.pallas import tpu as pltpu
from jax.experimental.pallas import tpu_sc as plsc

M = 262144
D = 128
B = 16384
LOG2B = 14

NC = 2
NS = 16
L = 16

TC_BLOCK = 1024


def _tc_norm_body(mem_ref, out_ref):
    x = mem_ref[...]
    normsq = jnp.sum(x * x, axis=1, keepdims=True)
    out_ref[...] = x * (1.0 / jnp.maximum(jnp.sqrt(normsq), 1e-12))


def _tc_normalize(mem, interpret=False):
    return pl.pallas_call(
        _tc_norm_body,
        grid=(M // TC_BLOCK,),
        in_specs=[pl.BlockSpec((TC_BLOCK, D), lambda i: (i, 0))],
        out_specs=pl.BlockSpec((TC_BLOCK, D), lambda i: (i, 0)),
        out_shape=jax.ShapeDtypeStruct((M, D), jnp.float32),
        compiler_params=pltpu.CompilerParams(
            dimension_semantics=("arbitrary",)),
        interpret=interpret,
    )(mem)


EPT = B // NS
G = 128
NG_ACC = EPT // G
HALF = B // 2
DUMP = HALF
S_ROWS = HALF + 8
COMB_ROWS = B + 8


def _mesh():
    return plsc.VectorSubcoreMesh(
        core_axis_name="c", subcore_axis_name="s", num_cores=NC,
        num_subcores=NS,
    )


def _rsqrt16(x):
    i = plsc.bitcast(x, jnp.int32)
    i = jnp.int32(0x5F3759DF) - (i >> 1)
    y = plsc.bitcast(i, jnp.float32)
    for _ in range(3):
        y = y * (1.5 - 0.5 * x * y * y)
    return y


PAD = EPT + L


def _acc_body(sidx_hbm, order_hbm, val_hbm, zeros_hbm, comb_hbm,
              S, sidx_v, order_v, slots_f, order_f, tgt_f,
              slots2, order2, tgt2, vbuf, cbuf):
    c = lax.axis_index("c")
    s = lax.axis_index("s")

    pltpu.sync_copy(sidx_hbm, sidx_v)
    pltpu.sync_copy(order_hbm.at[pl.ds(s * EPT, EPT)], order_v)

    for g in range(HALF // G // NS):
        pltpu.sync_copy(zeros_hbm,
                        S.at[pl.ds(s * (HALF // NS) + g * G, G)])

    @pl.when(s == 0)
    def _():
        pltpu.sync_copy(zeros_hbm.at[pl.ds(0, S_ROWS - HALF)],
                        S.at[pl.ds(HALF, S_ROWS - HALF)])

    def pad_body(v, _):
        slots_f[pl.ds(v * L, L)] = jnp.full((L,), DUMP, jnp.int32)
        order_f[pl.ds(v * L, L)] = jnp.zeros((L,), jnp.int32)
        tgt_f[pl.ds(v * L, L)] = jnp.full((L,), B, jnp.int32)
        return 0

    lax.fori_loop(0, PAD // L, pad_body, 0)

    def slot_body(v, cnt):
        pos_g = s * EPT + v * L + lax.iota(jnp.int32, L)
        x = sidx_v[pl.ds(s * EPT + v * L, L)]
        ov = order_v[pl.ds(v * L, L)]
        rep = jnp.zeros((L,), jnp.int32)
        for k in (1 << p for p in reversed(range(LOG2B))):
            cand = rep + jnp.int32(k)
            below = plsc.load_gather(sidx_v, [cand - 1]) < x
            rep = jnp.where(below, cand, rep)
        in_half = (rep >> 13) == c
        plsc.store_compressed(slots_f.at[pl.ds(cnt, L)],
                              rep - c * HALF, mask=in_half)
        plsc.store_compressed(order_f.at[pl.ds(cnt, L)], ov, mask=in_half)
        plsc.store_compressed(tgt_f.at[pl.ds(cnt, L)], pos_g, mask=in_half)
        return cnt + jnp.sum(in_half.astype(jnp.int32))

    k_cnt = lax.fori_loop(0, EPT // L, slot_body, jnp.int32(0))
    ng = (k_cnt + (G - 1)) >> 7

    def copy_body(v, _):
        r = v >> 3
        off = (v & 7) * L
        slots2[r, pl.ds(off, L)] = slots_f[pl.ds(v * L, L)]
        order2[r, pl.ds(off, L)] = order_f[pl.ds(v * L, L)]
        tgt2[r, pl.ds(off, L)] = tgt_f[pl.ds(v * L, L)]
        return 0

    lax.fori_loop(0, EPT // L, copy_body, 0)

    plsc.subcore_barrier()

    def acc_grp(g, _):
        pltpu.sync_copy(val_hbm.at[order2.at[g]], vbuf)
        pltpu.sync_copy(vbuf, S.at[slots2.at[g]], add=True)
        return 0

    lax.fori_loop(0, ng, acc_grp, 0)

    plsc.subcore_barrier()

    def out_grp(g, _):
        pltpu.sync_copy(S.at[slots2.at[g]], cbuf)
        pltpu.sync_copy(cbuf, comb_hbm.at[tgt2.at[g]])
        return 0

    lax.fori_loop(0, ng, out_grp, 0)


def _make_acc(interpret=False):
    return pl.kernel(
        _acc_body,
        out_type=jax.ShapeDtypeStruct((COMB_ROWS, D), jnp.float32),
        mesh=_mesh(),
        compiler_params=pltpu.CompilerParams(needs_layout_passes=False),
        interpret=interpret,
        scratch_types=[
            pltpu.VMEM_SHARED((S_ROWS, D), jnp.float32),
            pltpu.VMEM((B,), jnp.int32),
            pltpu.VMEM((EPT,), jnp.int32),
            pltpu.VMEM((PAD,), jnp.int32),
            pltpu.VMEM((PAD,), jnp.int32),
            pltpu.VMEM((PAD,), jnp.int32),
            pltpu.VMEM((NG_ACC, G), jnp.int32),
            pltpu.VMEM((NG_ACC, G), jnp.int32),
            pltpu.VMEM((NG_ACC, G), jnp.int32),
            pltpu.VMEM((G, D), jnp.float32),
            pltpu.VMEM((G, D), jnp.float32),
        ],
    )


EPW = B // (NC * NS)
NG_FIX = EPW // G


def _fix_rows(mbuf, cbuf):
    def row_body(r, _):
        us = []
        ns = jnp.zeros((L,), jnp.float32)
        for k in range(D // L):
            u = mbuf[r, pl.ds(k * L, L)] + cbuf[r, pl.ds(k * L, L)]
            us.append(u)
            ns = ns + u * u
        tot = jnp.full((L,), jnp.sum(ns), jnp.float32)
        inv = _rsqrt16(jnp.maximum(tot, 1e-24))
        for k in range(D // L):
            mbuf[r, pl.ds(k * L, L)] = us[k] * inv
        return 0

    lax.fori_loop(0, G, row_body, 0, unroll=2)


def _fix_body(mem_hbm, sidx2d_hbm, comb_hbm, out_ref, sidx_sc,
              mb0, cb0, mb1, cb1, sg0, sg1, ss0, ss1):
    c = lax.axis_index("c")
    s = lax.axis_index("s")
    wid = s * NC + c
    base = wid * EPW

    pltpu.sync_copy(sidx2d_hbm.at[pl.ds(wid * NG_FIX, NG_FIX)], sidx_sc)

    mbufs, cbufs = [mb0, mb1], [cb0, cb1]
    gsem, ssem = [sg0, sg1], [ss0, ss1]

    def gathers(g):
        b = g % 2
        d1 = pltpu.async_copy(mem_hbm.at[sidx_sc.at[g]], mbufs[b], gsem[b])
        d2 = pltpu.async_copy(comb_hbm.at[pl.ds(base + g * G, G)],
                              cbufs[b], gsem[b])
        return d1, d2

    scat = [None, None]
    pending = gathers(0)
    for g in range(NG_FIX):
        nxt = None
        if g + 1 < NG_FIX:
            nb = (g + 1) % 2
            if scat[nb] is not None:
                scat[nb].wait()
                scat[nb] = None
            nxt = gathers(g + 1)
        pending[0].wait()
        pending[1].wait()
        b = g % 2
        _fix_rows(mbufs[b], cbufs[b])
        scat[b] = pltpu.async_copy(mbufs[b], out_ref.at[sidx_sc.at[g]],
                                   ssem[b])
        pending = nxt

    for sd in scat:
        if sd is not None:
            sd.wait()


def _make_fix(interpret=False):
    return pl.kernel(
        _fix_body,
        mesh=_mesh(),
        compiler_params=pltpu.CompilerParams(needs_layout_passes=False),
        interpret=interpret,
        scratch_types=[
            pltpu.VMEM((NG_FIX, G), jnp.int32),
            pltpu.VMEM((G, D), jnp.float32),
            pltpu.VMEM((G, D), jnp.float32),
            pltpu.VMEM((G, D), jnp.float32),
            pltpu.VMEM((G, D), jnp.float32),
            pltpu.SemaphoreType.DMA,
            pltpu.SemaphoreType.DMA,
            pltpu.SemaphoreType.DMA,
            pltpu.SemaphoreType.DMA,
        ],
    )


def _kernel_impl(mem, idx, val, interpret=False):
    idx32 = idx.astype(jnp.int32)
    sidx, order = lax.sort_key_val(idx32, jnp.arange(B, dtype=jnp.int32))
    sidx2d = sidx.reshape(B // G, G)
    zeros = jnp.zeros((G, D), jnp.float32)

    out1 = _tc_normalize(mem, interpret)
    comb = _make_acc(interpret)(sidx, order, val, zeros)

    oref = jax.new_ref(out1)
    _make_fix(interpret)(mem, sidx2d, comb, oref)
    return oref[...]


@jax.jit
def kernel(mem, idx, val):
    return _kernel_impl(mem, idx, val)


# speedup vs baseline: 5.5695x; 1.5369x over previous
import jax
import jax.numpy as jnp
from jax import lax
from jax.experimental import pallas as pl
from jax.experimental.pallas import tpu as pltpu
from jax.experimental.pallas import tpu_sc as plsc

M = 262144
D = 128
B = 16384
LOG2B = 14

NC = 2
NS = 16
L = 16

TC_BLOCK = 4096


def _tc_norm_body(mem_ref, out_ref):
    x = mem_ref[...]
    normsq = jnp.sum(x * x, axis=1, keepdims=True)
    out_ref[...] = x * (1.0 / jnp.maximum(jnp.sqrt(normsq), 1e-12))


def _tc_normalize(mem, interpret=False):
    return pl.pallas_call(
        _tc_norm_body,
        grid=(M // TC_BLOCK,),
        in_specs=[pl.BlockSpec((TC_BLOCK, D), lambda i: (i, 0))],
        out_specs=pl.BlockSpec((TC_BLOCK, D), lambda i: (i, 0)),
        out_shape=jax.ShapeDtypeStruct((M, D), jnp.float32),
        compiler_params=pltpu.CompilerParams(
            dimension_semantics=("arbitrary",)),
        interpret=interpret,
    )(mem)


EPT = B // NS
G = 128
NG_ACC = EPT // G
HALF = B // 2
DUMP = HALF
S_ROWS = HALF + 8
COMB_ROWS = B + 8


def _mesh():
    return plsc.VectorSubcoreMesh(
        core_axis_name="c", subcore_axis_name="s", num_cores=NC,
        num_subcores=NS,
    )


def _rsqrt16(x):
    i = plsc.bitcast(x, jnp.int32)
    i = jnp.int32(0x5F3759DF) - (i >> 1)
    y = plsc.bitcast(i, jnp.float32)
    for _ in range(3):
        y = y * (1.5 - 0.5 * x * y * y)
    return y


PAD = EPT + L


def _acc_body(sidx_hbm, order_hbm, val_hbm, zeros_hbm, comb_hbm,
              S, sidx_v, order_v, slots_f, order_f, tgt_f,
              slots2, order2, tgt2, vbuf, cbuf):
    c = lax.axis_index("c")
    s = lax.axis_index("s")

    pltpu.sync_copy(sidx_hbm, sidx_v)
    pltpu.sync_copy(order_hbm.at[pl.ds(s * EPT, EPT)], order_v)

    for g in range(HALF // G // NS):
        pltpu.sync_copy(zeros_hbm,
                        S.at[pl.ds(s * (HALF // NS) + g * G, G)])

    @pl.when(s == 0)
    def _():
        pltpu.sync_copy(zeros_hbm.at[pl.ds(0, S_ROWS - HALF)],
                        S.at[pl.ds(HALF, S_ROWS - HALF)])

    def pad_body(v, _):
        slots_f[pl.ds(v * L, L)] = jnp.full((L,), DUMP, jnp.int32)
        order_f[pl.ds(v * L, L)] = jnp.zeros((L,), jnp.int32)
        tgt_f[pl.ds(v * L, L)] = jnp.full((L,), B, jnp.int32)
        return 0

    lax.fori_loop(0, PAD // L, pad_body, 0)

    def slot_body(v, cnt):
        pos_g = s * EPT + v * L + lax.iota(jnp.int32, L)
        x = sidx_v[pl.ds(s * EPT + v * L, L)]
        ov = order_v[pl.ds(v * L, L)]
        rep = jnp.zeros((L,), jnp.int32)
        for k in (1 << p for p in reversed(range(LOG2B))):
            cand = rep + jnp.int32(k)
            below = plsc.load_gather(sidx_v, [cand - 1]) < x
            rep = jnp.where(below, cand, rep)
        in_half = (rep >> 13) == c
        plsc.store_compressed(slots_f.at[pl.ds(cnt, L)],
                              rep - c * HALF, mask=in_half)
        plsc.store_compressed(order_f.at[pl.ds(cnt, L)], ov, mask=in_half)
        plsc.store_compressed(tgt_f.at[pl.ds(cnt, L)], pos_g, mask=in_half)
        return cnt + jnp.sum(in_half.astype(jnp.int32))

    k_cnt = lax.fori_loop(0, EPT // L, slot_body, jnp.int32(0))
    ng = (k_cnt + (G - 1)) >> 7

    def copy_body(v, _):
        r = v >> 3
        off = (v & 7) * L
        slots2[r, pl.ds(off, L)] = slots_f[pl.ds(v * L, L)]
        order2[r, pl.ds(off, L)] = order_f[pl.ds(v * L, L)]
        tgt2[r, pl.ds(off, L)] = tgt_f[pl.ds(v * L, L)]
        return 0

    lax.fori_loop(0, EPT // L, copy_body, 0)

    plsc.subcore_barrier()

    def acc_grp(g, _):
        pltpu.sync_copy(val_hbm.at[order2.at[g]], vbuf)
        pltpu.sync_copy(vbuf, S.at[slots2.at[g]], add=True)
        return 0

    lax.fori_loop(0, ng, acc_grp, 0)

    plsc.subcore_barrier()

    def out_grp(g, _):
        pltpu.sync_copy(S.at[slots2.at[g]], cbuf)
        pltpu.sync_copy(cbuf, comb_hbm.at[tgt2.at[g]])
        return 0

    lax.fori_loop(0, ng, out_grp, 0)


def _make_acc(interpret=False):
    return pl.kernel(
        _acc_body,
        out_type=jax.ShapeDtypeStruct((COMB_ROWS, D), jnp.float32),
        mesh=_mesh(),
        compiler_params=pltpu.CompilerParams(needs_layout_passes=False),
        interpret=interpret,
        scratch_types=[
            pltpu.VMEM_SHARED((S_ROWS, D), jnp.float32),
            pltpu.VMEM((B,), jnp.int32),
            pltpu.VMEM((EPT,), jnp.int32),
            pltpu.VMEM((PAD,), jnp.int32),
            pltpu.VMEM((PAD,), jnp.int32),
            pltpu.VMEM((PAD,), jnp.int32),
            pltpu.VMEM((NG_ACC, G), jnp.int32),
            pltpu.VMEM((NG_ACC, G), jnp.int32),
            pltpu.VMEM((NG_ACC, G), jnp.int32),
            pltpu.VMEM((G, D), jnp.float32),
            pltpu.VMEM((G, D), jnp.float32),
        ],
    )


EPW = B // (NC * NS)
NG_FIX = EPW // G


def _fix_rows(mbuf, cbuf):
    def row_body(r, _):
        us = []
        ns = jnp.zeros((L,), jnp.float32)
        for k in range(D // L):
            u = mbuf[r, pl.ds(k * L, L)] + cbuf[r, pl.ds(k * L, L)]
            us.append(u)
            ns = ns + u * u
        tot = jnp.full((L,), jnp.sum(ns), jnp.float32)
        inv = _rsqrt16(jnp.maximum(tot, 1e-24))
        for k in range(D // L):
            mbuf[r, pl.ds(k * L, L)] = us[k] * inv
        return 0

    lax.fori_loop(0, G, row_body, 0, unroll=2)


def _fix_body(mem_hbm, sidx2d_hbm, comb_hbm, out_ref, sidx_sc,
              mb0, cb0, mb1, cb1, sg0, sg1, ss0, ss1):
    c = lax.axis_index("c")
    s = lax.axis_index("s")
    wid = s * NC + c
    base = wid * EPW

    pltpu.sync_copy(sidx2d_hbm.at[pl.ds(wid * NG_FIX, NG_FIX)], sidx_sc)

    mbufs, cbufs = [mb0, mb1], [cb0, cb1]
    gsem, ssem = [sg0, sg1], [ss0, ss1]

    def gathers(g):
        b = g % 2
        d1 = pltpu.async_copy(mem_hbm.at[sidx_sc.at[g]], mbufs[b], gsem[b])
        d2 = pltpu.async_copy(comb_hbm.at[pl.ds(base + g * G, G)],
                              cbufs[b], gsem[b])
        return d1, d2

    scat = [None, None]
    pending = gathers(0)
    for g in range(NG_FIX):
        nxt = None
        if g + 1 < NG_FIX:
            nb = (g + 1) % 2
            if scat[nb] is not None:
                scat[nb].wait()
                scat[nb] = None
            nxt = gathers(g + 1)
        pending[0].wait()
        pending[1].wait()
        b = g % 2
        _fix_rows(mbufs[b], cbufs[b])
        scat[b] = pltpu.async_copy(mbufs[b], out_ref.at[sidx_sc.at[g]],
                                   ssem[b])
        pending = nxt

    for sd in scat:
        if sd is not None:
            sd.wait()


def _make_fix(interpret=False):
    return pl.kernel(
        _fix_body,
        mesh=_mesh(),
        compiler_params=pltpu.CompilerParams(needs_layout_passes=False),
        interpret=interpret,
        scratch_types=[
            pltpu.VMEM((NG_FIX, G), jnp.int32),
            pltpu.VMEM((G, D), jnp.float32),
            pltpu.VMEM((G, D), jnp.float32),
            pltpu.VMEM((G, D), jnp.float32),
            pltpu.VMEM((G, D), jnp.float32),
            pltpu.SemaphoreType.DMA,
            pltpu.SemaphoreType.DMA,
            pltpu.SemaphoreType.DMA,
            pltpu.SemaphoreType.DMA,
        ],
    )


def _kernel_impl(mem, idx, val, interpret=False):
    idx32 = idx.astype(jnp.int32)
    sidx, order = lax.sort_key_val(idx32, jnp.arange(B, dtype=jnp.int32))
    sidx2d = sidx.reshape(B // G, G)
    zeros = jnp.zeros((G, D), jnp.float32)

    out1 = _tc_normalize(mem, interpret)
    comb = _make_acc(interpret)(sidx, order, val, zeros)

    oref = jax.new_ref(out1)
    _make_fix(interpret)(mem, sidx2d, comb, oref)
    return oref[...]


@jax.jit
def kernel(mem, idx, val):
    return _kernel_impl(mem, idx, val)


# speedup vs baseline: 6.0846x; 1.0925x over previous
import jax
import jax.numpy as jnp
from jax import lax
from jax.experimental import pallas as pl
from jax.experimental.pallas import tpu as pltpu
from jax.experimental.pallas import tpu_sc as plsc

M = 262144
D = 128
B = 16384
LOG2B = 14

NC = 2
NS = 16
L = 16

TC_BLOCK = 8192


def _tc_norm_body(mem_ref, out_ref):
    x = mem_ref[...]
    normsq = jnp.sum(x * x, axis=1, keepdims=True)
    out_ref[...] = x * (1.0 / jnp.maximum(jnp.sqrt(normsq), 1e-12))


def _tc_normalize(mem, interpret=False):
    return pl.pallas_call(
        _tc_norm_body,
        grid=(M // TC_BLOCK,),
        in_specs=[pl.BlockSpec((TC_BLOCK, D), lambda i: (i, 0))],
        out_specs=pl.BlockSpec((TC_BLOCK, D), lambda i: (i, 0)),
        out_shape=jax.ShapeDtypeStruct((M, D), jnp.float32),
        compiler_params=pltpu.CompilerParams(
            dimension_semantics=("arbitrary",)),
        interpret=interpret,
    )(mem)


EPT = B // NS
G = 128
NG_ACC = EPT // G
HALF = B // 2
DUMP = HALF
S_ROWS = HALF + 8
COMB_ROWS = B + 8


def _mesh():
    return plsc.VectorSubcoreMesh(
        core_axis_name="c", subcore_axis_name="s", num_cores=NC,
        num_subcores=NS,
    )


def _rsqrt16(x):
    i = plsc.bitcast(x, jnp.int32)
    i = jnp.int32(0x5F3759DF) - (i >> 1)
    y = plsc.bitcast(i, jnp.float32)
    for _ in range(3):
        y = y * (1.5 - 0.5 * x * y * y)
    return y


PAD = EPT + L


def _acc_body(sidx_hbm, order_hbm, val_hbm, zeros_hbm, comb_hbm,
              S, sidx_v, order_v, slots_f, order_f, tgt_f,
              slots2, order2, tgt2, vbuf, cbuf):
    c = lax.axis_index("c")
    s = lax.axis_index("s")

    pltpu.sync_copy(sidx_hbm, sidx_v)
    pltpu.sync_copy(order_hbm.at[pl.ds(s * EPT, EPT)], order_v)

    for g in range(HALF // G // NS):
        pltpu.sync_copy(zeros_hbm,
                        S.at[pl.ds(s * (HALF // NS) + g * G, G)])

    @pl.when(s == 0)
    def _():
        pltpu.sync_copy(zeros_hbm.at[pl.ds(0, S_ROWS - HALF)],
                        S.at[pl.ds(HALF, S_ROWS - HALF)])

    def pad_body(v, _):
        slots_f[pl.ds(v * L, L)] = jnp.full((L,), DUMP, jnp.int32)
        order_f[pl.ds(v * L, L)] = jnp.zeros((L,), jnp.int32)
        tgt_f[pl.ds(v * L, L)] = jnp.full((L,), B, jnp.int32)
        return 0

    lax.fori_loop(0, PAD // L, pad_body, 0)

    def slot_body(v, cnt):
        pos_g = s * EPT + v * L + lax.iota(jnp.int32, L)
        x = sidx_v[pl.ds(s * EPT + v * L, L)]
        ov = order_v[pl.ds(v * L, L)]
        rep = jnp.zeros((L,), jnp.int32)
        for k in (1 << p for p in reversed(range(LOG2B))):
            cand = rep + jnp.int32(k)
            below = plsc.load_gather(sidx_v, [cand - 1]) < x
            rep = jnp.where(below, cand, rep)
        in_half = (rep >> 13) == c
        plsc.store_compressed(slots_f.at[pl.ds(cnt, L)],
                              rep - c * HALF, mask=in_half)
        plsc.store_compressed(order_f.at[pl.ds(cnt, L)], ov, mask=in_half)
        plsc.store_compressed(tgt_f.at[pl.ds(cnt, L)], pos_g, mask=in_half)
        return cnt + jnp.sum(in_half.astype(jnp.int32))

    k_cnt = lax.fori_loop(0, EPT // L, slot_body, jnp.int32(0))
    ng = (k_cnt + (G - 1)) >> 7

    def copy_body(v, _):
        r = v >> 3
        off = (v & 7) * L
        slots2[r, pl.ds(off, L)] = slots_f[pl.ds(v * L, L)]
        order2[r, pl.ds(off, L)] = order_f[pl.ds(v * L, L)]
        tgt2[r, pl.ds(off, L)] = tgt_f[pl.ds(v * L, L)]
        return 0

    lax.fori_loop(0, EPT // L, copy_body, 0)

    plsc.subcore_barrier()

    def acc_grp(g, _):
        pltpu.sync_copy(val_hbm.at[order2.at[g]], vbuf)
        pltpu.sync_copy(vbuf, S.at[slots2.at[g]], add=True)
        return 0

    lax.fori_loop(0, ng, acc_grp, 0)

    plsc.subcore_barrier()

    def out_grp(g, _):
        pltpu.sync_copy(S.at[slots2.at[g]], cbuf)
        pltpu.sync_copy(cbuf, comb_hbm.at[tgt2.at[g]])
        return 0

    lax.fori_loop(0, ng, out_grp, 0)


def _make_acc(interpret=False):
    return pl.kernel(
        _acc_body,
        out_type=jax.ShapeDtypeStruct((COMB_ROWS, D), jnp.float32),
        mesh=_mesh(),
        compiler_params=pltpu.CompilerParams(needs_layout_passes=False),
        interpret=interpret,
        scratch_types=[
            pltpu.VMEM_SHARED((S_ROWS, D), jnp.float32),
            pltpu.VMEM((B,), jnp.int32),
            pltpu.VMEM((EPT,), jnp.int32),
            pltpu.VMEM((PAD,), jnp.int32),
            pltpu.VMEM((PAD,), jnp.int32),
            pltpu.VMEM((PAD,), jnp.int32),
            pltpu.VMEM((NG_ACC, G), jnp.int32),
            pltpu.VMEM((NG_ACC, G), jnp.int32),
            pltpu.VMEM((NG_ACC, G), jnp.int32),
            pltpu.VMEM((G, D), jnp.float32),
            pltpu.VMEM((G, D), jnp.float32),
        ],
    )


EPW = B // (NC * NS)
NG_FIX = EPW // G


def _fix_rows(mbuf, cbuf):
    def row_body(r, _):
        us = []
        ns = jnp.zeros((L,), jnp.float32)
        for k in range(D // L):
            u = mbuf[r, pl.ds(k * L, L)] + cbuf[r, pl.ds(k * L, L)]
            us.append(u)
            ns = ns + u * u
        tot = jnp.full((L,), jnp.sum(ns), jnp.float32)
        inv = _rsqrt16(jnp.maximum(tot, 1e-24))
        for k in range(D // L):
            mbuf[r, pl.ds(k * L, L)] = us[k] * inv
        return 0

    lax.fori_loop(0, G, row_body, 0, unroll=2)


def _fix_body(mem_hbm, sidx2d_hbm, comb_hbm, out_ref, sidx_sc,
              mb0, cb0, mb1, cb1, sg0, sg1, ss0, ss1):
    c = lax.axis_index("c")
    s = lax.axis_index("s")
    wid = s * NC + c
    base = wid * EPW

    pltpu.sync_copy(sidx2d_hbm.at[pl.ds(wid * NG_FIX, NG_FIX)], sidx_sc)

    mbufs, cbufs = [mb0, mb1], [cb0, cb1]
    gsem, ssem = [sg0, sg1], [ss0, ss1]

    def gathers(g):
        b = g % 2
        d1 = pltpu.async_copy(mem_hbm.at[sidx_sc.at[g]], mbufs[b], gsem[b])
        d2 = pltpu.async_copy(comb_hbm.at[pl.ds(base + g * G, G)],
                              cbufs[b], gsem[b])
        return d1, d2

    scat = [None, None]
    pending = gathers(0)
    for g in range(NG_FIX):
        nxt = None
        if g + 1 < NG_FIX:
            nb = (g + 1) % 2
            if scat[nb] is not None:
                scat[nb].wait()
                scat[nb] = None
            nxt = gathers(g + 1)
        pending[0].wait()
        pending[1].wait()
        b = g % 2
        _fix_rows(mbufs[b], cbufs[b])
        scat[b] = pltpu.async_copy(mbufs[b], out_ref.at[sidx_sc.at[g]],
                                   ssem[b])
        pending = nxt

    for sd in scat:
        if sd is not None:
            sd.wait()


def _make_fix(interpret=False):
    return pl.kernel(
        _fix_body,
        mesh=_mesh(),
        compiler_params=pltpu.CompilerParams(needs_layout_passes=False),
        interpret=interpret,
        scratch_types=[
            pltpu.VMEM((NG_FIX, G), jnp.int32),
            pltpu.VMEM((G, D), jnp.float32),
            pltpu.VMEM((G, D), jnp.float32),
            pltpu.VMEM((G, D), jnp.float32),
            pltpu.VMEM((G, D), jnp.float32),
            pltpu.SemaphoreType.DMA,
            pltpu.SemaphoreType.DMA,
            pltpu.SemaphoreType.DMA,
            pltpu.SemaphoreType.DMA,
        ],
    )


def _kernel_impl(mem, idx, val, interpret=False):
    idx32 = idx.astype(jnp.int32)
    sidx, order = lax.sort_key_val(idx32, jnp.arange(B, dtype=jnp.int32))
    sidx2d = sidx.reshape(B // G, G)
    zeros = jnp.zeros((G, D), jnp.float32)

    out1 = _tc_normalize(mem, interpret)
    comb = _make_acc(interpret)(sidx, order, val, zeros)

    oref = jax.new_ref(out1)
    _make_fix(interpret)(mem, sidx2d, comb, oref)
    return oref[...]


@jax.jit
def kernel(mem, idx, val):
    return _kernel_impl(mem, idx, val)


# speedup vs baseline: 6.1806x; 1.0158x over previous
import jax
import jax.numpy as jnp
from jax import lax
from jax.experimental import pallas as pl
from jax.experimental.pallas import tpu as pltpu
from jax.experimental.pallas import tpu_sc as plsc

M = 262144
D = 128
B = 16384
LOG2B = 14

NC = 2
NS = 16
L = 16

TC_BLOCK = 16384


def _tc_norm_body(mem_ref, out_ref):
    x = mem_ref[...]
    normsq = jnp.sum(x * x, axis=1, keepdims=True)
    out_ref[...] = x * (1.0 / jnp.maximum(jnp.sqrt(normsq), 1e-12))


def _tc_normalize(mem, interpret=False):
    return pl.pallas_call(
        _tc_norm_body,
        grid=(M // TC_BLOCK,),
        in_specs=[pl.BlockSpec((TC_BLOCK, D), lambda i: (i, 0))],
        out_specs=pl.BlockSpec((TC_BLOCK, D), lambda i: (i, 0)),
        out_shape=jax.ShapeDtypeStruct((M, D), jnp.float32),
        compiler_params=pltpu.CompilerParams(
            dimension_semantics=("arbitrary",)),
        interpret=interpret,
    )(mem)


EPT = B // NS
G = 128
NG_ACC = EPT // G
HALF = B // 2
DUMP = HALF
S_ROWS = HALF + 8
COMB_ROWS = B + 8


def _mesh():
    return plsc.VectorSubcoreMesh(
        core_axis_name="c", subcore_axis_name="s", num_cores=NC,
        num_subcores=NS,
    )


def _rsqrt16(x):
    i = plsc.bitcast(x, jnp.int32)
    i = jnp.int32(0x5F3759DF) - (i >> 1)
    y = plsc.bitcast(i, jnp.float32)
    for _ in range(3):
        y = y * (1.5 - 0.5 * x * y * y)
    return y


PAD = EPT + L


def _acc_body(sidx_hbm, order_hbm, val_hbm, zeros_hbm, comb_hbm,
              S, sidx_v, order_v, slots_f, order_f, tgt_f,
              slots2, order2, tgt2, vbuf, cbuf):
    c = lax.axis_index("c")
    s = lax.axis_index("s")

    pltpu.sync_copy(sidx_hbm, sidx_v)
    pltpu.sync_copy(order_hbm.at[pl.ds(s * EPT, EPT)], order_v)

    for g in range(HALF // G // NS):
        pltpu.sync_copy(zeros_hbm,
                        S.at[pl.ds(s * (HALF // NS) + g * G, G)])

    @pl.when(s == 0)
    def _():
        pltpu.sync_copy(zeros_hbm.at[pl.ds(0, S_ROWS - HALF)],
                        S.at[pl.ds(HALF, S_ROWS - HALF)])

    def pad_body(v, _):
        slots_f[pl.ds(v * L, L)] = jnp.full((L,), DUMP, jnp.int32)
        order_f[pl.ds(v * L, L)] = jnp.zeros((L,), jnp.int32)
        tgt_f[pl.ds(v * L, L)] = jnp.full((L,), B, jnp.int32)
        return 0

    lax.fori_loop(0, PAD // L, pad_body, 0)

    def slot_body(v, cnt):
        pos_g = s * EPT + v * L + lax.iota(jnp.int32, L)
        x = sidx_v[pl.ds(s * EPT + v * L, L)]
        ov = order_v[pl.ds(v * L, L)]
        rep = jnp.zeros((L,), jnp.int32)
        for k in (1 << p for p in reversed(range(LOG2B))):
            cand = rep + jnp.int32(k)
            below = plsc.load_gather(sidx_v, [cand - 1]) < x
            rep = jnp.where(below, cand, rep)
        in_half = (rep >> 13) == c
        plsc.store_compressed(slots_f.at[pl.ds(cnt, L)],
                              rep - c * HALF, mask=in_half)
        plsc.store_compressed(order_f.at[pl.ds(cnt, L)], ov, mask=in_half)
        plsc.store_compressed(tgt_f.at[pl.ds(cnt, L)], pos_g, mask=in_half)
        return cnt + jnp.sum(in_half.astype(jnp.int32))

    k_cnt = lax.fori_loop(0, EPT // L, slot_body, jnp.int32(0))
    ng = (k_cnt + (G - 1)) >> 7

    def copy_body(v, _):
        r = v >> 3
        off = (v & 7) * L
        slots2[r, pl.ds(off, L)] = slots_f[pl.ds(v * L, L)]
        order2[r, pl.ds(off, L)] = order_f[pl.ds(v * L, L)]
        tgt2[r, pl.ds(off, L)] = tgt_f[pl.ds(v * L, L)]
        return 0

    lax.fori_loop(0, EPT // L, copy_body, 0)

    plsc.subcore_barrier()

    def acc_grp(g, _):
        pltpu.sync_copy(val_hbm.at[order2.at[g]], vbuf)
        pltpu.sync_copy(vbuf, S.at[slots2.at[g]], add=True)
        return 0

    lax.fori_loop(0, ng, acc_grp, 0)

    plsc.subcore_barrier()

    def out_grp(g, _):
        pltpu.sync_copy(S.at[slots2.at[g]], cbuf)
        pltpu.sync_copy(cbuf, comb_hbm.at[tgt2.at[g]])
        return 0

    lax.fori_loop(0, ng, out_grp, 0)


def _make_acc(interpret=False):
    return pl.kernel(
        _acc_body,
        out_type=jax.ShapeDtypeStruct((COMB_ROWS, D), jnp.float32),
        mesh=_mesh(),
        compiler_params=pltpu.CompilerParams(needs_layout_passes=False),
        interpret=interpret,
        scratch_types=[
            pltpu.VMEM_SHARED((S_ROWS, D), jnp.float32),
            pltpu.VMEM((B,), jnp.int32),
            pltpu.VMEM((EPT,), jnp.int32),
            pltpu.VMEM((PAD,), jnp.int32),
            pltpu.VMEM((PAD,), jnp.int32),
            pltpu.VMEM((PAD,), jnp.int32),
            pltpu.VMEM((NG_ACC, G), jnp.int32),
            pltpu.VMEM((NG_ACC, G), jnp.int32),
            pltpu.VMEM((NG_ACC, G), jnp.int32),
            pltpu.VMEM((G, D), jnp.float32),
            pltpu.VMEM((G, D), jnp.float32),
        ],
    )


EPW = B // (NC * NS)
NG_FIX = EPW // G


def _fix_rows(mbuf, cbuf):
    def row_body(r, _):
        us = []
        ns = jnp.zeros((L,), jnp.float32)
        for k in range(D // L):
            u = mbuf[r, pl.ds(k * L, L)] + cbuf[r, pl.ds(k * L, L)]
            us.append(u)
            ns = ns + u * u
        tot = jnp.full((L,), jnp.sum(ns), jnp.float32)
        inv = _rsqrt16(jnp.maximum(tot, 1e-24))
        for k in range(D // L):
            mbuf[r, pl.ds(k * L, L)] = us[k] * inv
        return 0

    lax.fori_loop(0, G, row_body, 0, unroll=2)


def _fix_body(mem_hbm, sidx2d_hbm, comb_hbm, out_ref, sidx_sc,
              mb0, cb0, mb1, cb1, sg0, sg1, ss0, ss1):
    c = lax.axis_index("c")
    s = lax.axis_index("s")
    wid = s * NC + c
    base = wid * EPW

    pltpu.sync_copy(sidx2d_hbm.at[pl.ds(wid * NG_FIX, NG_FIX)], sidx_sc)

    mbufs, cbufs = [mb0, mb1], [cb0, cb1]
    gsem, ssem = [sg0, sg1], [ss0, ss1]

    def gathers(g):
        b = g % 2
        d1 = pltpu.async_copy(mem_hbm.at[sidx_sc.at[g]], mbufs[b], gsem[b])
        d2 = pltpu.async_copy(comb_hbm.at[pl.ds(base + g * G, G)],
                              cbufs[b], gsem[b])
        return d1, d2

    scat = [None, None]
    pending = gathers(0)
    for g in range(NG_FIX):
        nxt = None
        if g + 1 < NG_FIX:
            nb = (g + 1) % 2
            if scat[nb] is not None:
                scat[nb].wait()
                scat[nb] = None
            nxt = gathers(g + 1)
        pending[0].wait()
        pending[1].wait()
        b = g % 2
        _fix_rows(mbufs[b], cbufs[b])
        scat[b] = pltpu.async_copy(mbufs[b], out_ref.at[sidx_sc.at[g]],
                                   ssem[b])
        pending = nxt

    for sd in scat:
        if sd is not None:
            sd.wait()


def _make_fix(interpret=False):
    return pl.kernel(
        _fix_body,
        mesh=_mesh(),
        compiler_params=pltpu.CompilerParams(needs_layout_passes=False),
        interpret=interpret,
        scratch_types=[
            pltpu.VMEM((NG_FIX, G), jnp.int32),
            pltpu.VMEM((G, D), jnp.float32),
            pltpu.VMEM((G, D), jnp.float32),
            pltpu.VMEM((G, D), jnp.float32),
            pltpu.VMEM((G, D), jnp.float32),
            pltpu.SemaphoreType.DMA,
            pltpu.SemaphoreType.DMA,
            pltpu.SemaphoreType.DMA,
            pltpu.SemaphoreType.DMA,
        ],
    )


def _kernel_impl(mem, idx, val, interpret=False):
    idx32 = idx.astype(jnp.int32)
    sidx, order = lax.sort_key_val(idx32, jnp.arange(B, dtype=jnp.int32))
    sidx2d = sidx.reshape(B // G, G)
    zeros = jnp.zeros((G, D), jnp.float32)

    out1 = _tc_normalize(mem, interpret)
    comb = _make_acc(interpret)(sidx, order, val, zeros)

    oref = jax.new_ref(out1)
    _make_fix(interpret)(mem, sidx2d, comb, oref)
    return oref[...]


@jax.jit
def kernel(mem, idx, val):
    return _kernel_impl(mem, idx, val)
